# baseline (speedup 1.0000x reference)
import functools

import jax
import jax.numpy as jnp
from jax import lax
from jax.experimental import pallas as pl
from jax.experimental.pallas import tpu as pltpu
from jax.experimental.pallas import tpu_sc as plsc

T_OUT = 2048


@functools.cache
def _make_unpack(N, D, B):
    info = plsc.get_sparse_core_info()
    NC, NS, L = info.num_cores, info.num_subcores, info.num_lanes
    NW = NC * NS
    PW = (B * T_OUT) // NW
    assert PW * NW == B * T_OUT and T_OUT % PW == 0
    CH = T_OUT // PW
    G = 32
    NG = PW // G
    NBUF = 3
    ZR = 16

    mesh = plsc.VectorSubcoreMesh(core_axis_name="c", subcore_axis_name="s")

    @functools.partial(
        pl.kernel,
        mesh=mesh,
        out_type=jax.ShapeDtypeStruct((B * T_OUT, D), jnp.float32),
        scratch_types=[
            pltpu.VMEM((PW,), jnp.int32),
            pltpu.VMEM((L,), jnp.int32),
            *[pltpu.VMEM((G, D), jnp.float32) for _ in range(NBUF)],
            pltpu.VMEM((ZR, D), jnp.float32),
            pltpu.VMEM_SHARED((G, D), jnp.float32),
            *[pltpu.SemaphoreType.DMA for _ in range(2 * NBUF + 1)],
        ],
    )
    def unpack(data_hbm, len_hbm, out_hbm, idx_v, len_v, *rest):
        bufs = rest[:NBUF]
        sbuf = rest[NBUF]
        zbuf = rest[NBUF + 1]
        gsems = rest[NBUF + 2:2 * NBUF + 2]
        wsems = rest[2 * NBUF + 2:3 * NBUF + 2]
        zsem = rest[3 * NBUF + 2]
        wid = lax.axis_index("s") * NC + lax.axis_index("c")
        b = wid // CH
        t0 = (wid % CH) * PW
        row0 = wid * PW

        len_v[...] = jnp.zeros((L,), jnp.int32)
        pltpu.sync_copy(len_hbm, len_v.at[pl.ds(0, B)])
        lanes = lax.iota(jnp.int32, L)
        lv = len_v[...]
        lens = [lv[j] for j in range(B)]
        len_b = lens[0] * 0
        for j in range(B):
            len_b = jnp.where(b == j, lens[j], len_b)
        v = jnp.clip(len_b - t0, 0, PW)

        def idx_fill(s):
            t_vec = t0 + s * L + lanes
            acc = jnp.zeros((L,), jnp.int32)
            for lj in lens:
                acc = acc + jnp.minimum(t_vec, lj)
            idx_v[pl.ds(s * L, L)] = jnp.minimum(acc + b, N - 1)

        def gather(g, p):
            return pltpu.make_async_copy(
                data_hbm.at[idx_v.at[pl.ds(g * G, G)]], bufs[p], gsems[p]
            )

        def write(g, p):
            return pltpu.make_async_copy(
                bufs[p], out_hbm.at[pl.ds(row0 + g * G, G)], wsems[p]
            )

        NPRO = min(NBUF, NG)
        for s in range((NPRO * G + L - 1) // L):
            idx_fill(s)
        for g in range(NPRO):
            @pl.when(g * G < v)
            def _(g=g):
                gather(g, g % NBUF).start()

        sid = lax.axis_index("s")

        @pl.when(sid == 0)
        def _():
            def zrow(i, carry):
                for c in range(D // L):
                    sbuf[i, pl.ds(c * L, L)] = jnp.zeros((L,), jnp.float32)
                return carry

            lax.fori_loop(0, ZR, zrow, 0)
            for q in range(G // ZR):
                pltpu.sync_copy(sbuf, zbuf.at[pl.ds(q * ZR, ZR)])

        plsc.subcore_barrier()

        for g in range(NG):
            @pl.when(g * G >= v)
            def _(g=g):
                pltpu.make_async_copy(
                    zbuf, out_hbm.at[pl.ds(row0 + g * G, G)], zsem
                ).start()

        for s in range((NPRO * G + L - 1) // L, PW // L):
            idx_fill(s)

        for g in range(NG):
            p = g % NBUF

            @pl.when(g * G < v)
            def _(g=g, p=p):
                gather(g, p).wait()

                @pl.when(v < (g + 1) * G)
                def _():
                    def zfix(i, carry):
                        for c in range(D // L):
                            bufs[p][i, pl.ds(c * L, L)] = jnp.zeros(
                                (L,), jnp.float32)
                        return carry

                    lax.fori_loop(v - g * G, G, zfix, 0)

                write(g, p).start()

            if g + NBUF < NG:
                @pl.when((g + NBUF) * G < v)
                def _(g=g, p=p):
                    write(g, p).wait()
                    gather(g + NBUF, p).start()

        for g in range(NG):
            p = g % NBUF
            in_loop = (g + NBUF) * G < v if g + NBUF < NG else False

            @pl.when((g * G < v) & jnp.logical_not(in_loop))
            def _(g=g, p=p):
                write(g, p).wait()

            @pl.when(g * G >= v)
            def _(g=g):
                pltpu.make_async_copy(
                    zbuf, out_hbm.at[pl.ds(row0 + g * G, G)], zsem
                ).wait()

    return unpack


def kernel(data, lengths):
    N, D = data.shape
    B = lengths.shape[0]
    out = _make_unpack(N, D, B)(data, lengths.astype(jnp.int32))
    return out.reshape(B, T_OUT, D), lengths


# speedup vs baseline: 1.0166x; 1.0166x over previous
import functools

import jax
import jax.numpy as jnp
from jax import lax
from jax.experimental import pallas as pl
from jax.experimental.pallas import tpu as pltpu
from jax.experimental.pallas import tpu_sc as plsc

T_OUT = 2048


@functools.cache
def _make_unpack(N, D, B):
    info = plsc.get_sparse_core_info()
    NC, NS, L = info.num_cores, info.num_subcores, info.num_lanes
    NW = NC * NS
    PW = (B * T_OUT) // NW
    assert PW * NW == B * T_OUT and T_OUT % PW == 0
    CH = T_OUT // PW
    G = 32
    NG = PW // G
    NBUF = 3
    ZR = 16

    mesh = plsc.VectorSubcoreMesh(core_axis_name="c", subcore_axis_name="s")

    @functools.partial(
        pl.kernel,
        mesh=mesh,
        out_type=jax.ShapeDtypeStruct((B * T_OUT, D), jnp.float32),
        scratch_types=[
            pltpu.VMEM((PW,), jnp.int32),
            pltpu.VMEM((L,), jnp.int32),
            *[pltpu.VMEM((G, D), jnp.float32) for _ in range(NBUF)],
            pltpu.VMEM((ZR, D), jnp.float32),
            *[pltpu.SemaphoreType.DMA for _ in range(2 * NBUF + 1)],
        ],
    )
    def unpack(data_hbm, len_hbm, out_hbm, idx_v, len_v, *rest):
        bufs = rest[:NBUF]
        zbuf = rest[NBUF]
        gsems = rest[NBUF + 1:2 * NBUF + 1]
        wsems = rest[2 * NBUF + 1:3 * NBUF + 1]
        zsem = rest[3 * NBUF + 1]
        wid = lax.axis_index("s") * NC + lax.axis_index("c")
        b = wid // CH
        t0 = (wid % CH) * PW
        row0 = wid * PW

        len_v[...] = jnp.zeros((L,), jnp.int32)
        pltpu.sync_copy(len_hbm, len_v.at[pl.ds(0, B)])
        lanes = lax.iota(jnp.int32, L)
        lv = len_v[...]
        lens = [lv[j] for j in range(B)]
        len_b = lens[0] * 0
        for j in range(B):
            len_b = jnp.where(b == j, lens[j], len_b)
        v = jnp.clip(len_b - t0, 0, PW)

        def idx_fill(s):
            t_vec = t0 + s * L + lanes
            acc = jnp.zeros((L,), jnp.int32)
            for lj in lens:
                acc = acc + jnp.minimum(t_vec, lj)
            idx_v[pl.ds(s * L, L)] = jnp.minimum(acc + b, N - 1)

        def gather(g, p):
            return pltpu.make_async_copy(
                data_hbm.at[idx_v.at[pl.ds(g * G, G)]], bufs[p], gsems[p]
            )

        def write(g, p):
            return pltpu.make_async_copy(
                bufs[p], out_hbm.at[pl.ds(row0 + g * G, G)], wsems[p]
            )

        NPRO = min(NBUF, NG)
        for s in range((NPRO * G + L - 1) // L):
            idx_fill(s)
        for g in range(NPRO):
            @pl.when(g * G < v)
            def _(g=g):
                gather(g, g % NBUF).start()

        def zrow(i, carry):
            for c in range(D // L):
                zbuf[i, pl.ds(c * L, L)] = jnp.zeros((L,), jnp.float32)
            return carry

        lax.fori_loop(0, ZR, zrow, 0)

        for g in range(NG):
            @pl.when(g * G >= v)
            def _(g=g):
                for q in range(G // ZR):
                    pltpu.make_async_copy(
                        zbuf,
                        out_hbm.at[pl.ds(row0 + g * G + q * ZR, ZR)],
                        zsem,
                    ).start()

        for s in range((NPRO * G + L - 1) // L, PW // L):
            idx_fill(s)

        for g in range(NG):
            p = g % NBUF

            @pl.when(g * G < v)
            def _(g=g, p=p):
                gather(g, p).wait()

                @pl.when(v < (g + 1) * G)
                def _():
                    def zfix(i, carry):
                        for c in range(D // L):
                            bufs[p][i, pl.ds(c * L, L)] = jnp.zeros(
                                (L,), jnp.float32)
                        return carry

                    lax.fori_loop(v - g * G, G, zfix, 0)

                write(g, p).start()

            if g + NBUF < NG:
                @pl.when((g + NBUF) * G < v)
                def _(g=g, p=p):
                    write(g, p).wait()
                    gather(g + NBUF, p).start()

        for g in range(NG):
            p = g % NBUF
            in_loop = (g + NBUF) * G < v if g + NBUF < NG else False

            @pl.when((g * G < v) & jnp.logical_not(in_loop))
            def _(g=g, p=p):
                write(g, p).wait()

            @pl.when(g * G >= v)
            def _(g=g):
                for q in range(G // ZR):
                    pltpu.make_async_copy(
                        zbuf,
                        out_hbm.at[pl.ds(row0 + g * G + q * ZR, ZR)],
                        zsem,
                    ).wait()

    return unpack


def kernel(data, lengths):
    N, D = data.shape
    B = lengths.shape[0]
    out = _make_unpack(N, D, B)(data, lengths.astype(jnp.int32))
    return out.reshape(B, T_OUT, D), lengths


# speedup vs baseline: 1.0596x; 1.0423x over previous
import functools

import jax
import jax.numpy as jnp
from jax import lax
from jax.experimental import pallas as pl
from jax.experimental.pallas import tpu as pltpu
from jax.experimental.pallas import tpu_sc as plsc

T_OUT = 2048


@functools.cache
def _make_unpack(N, D, B):
    info = plsc.get_sparse_core_info()
    NC, NS, L = info.num_cores, info.num_subcores, info.num_lanes
    NW = NC * NS
    G = 32
    NGT = (B * T_OUT) // G
    KG = NGT // NW
    assert KG * NW == NGT and T_OUT % G == 0
    GB = T_OUT // G
    NBUF = 2

    mesh = plsc.VectorSubcoreMesh(core_axis_name="c", subcore_axis_name="s")

    @functools.partial(
        pl.kernel,
        mesh=mesh,
        out_type=jax.ShapeDtypeStruct((B * T_OUT, D), jnp.float32),
        scratch_types=[
            pltpu.VMEM((KG * G,), jnp.int32),
            pltpu.VMEM((L,), jnp.int32),
            *[pltpu.VMEM((G, D), jnp.float32) for _ in range(NBUF)],
            pltpu.VMEM((G, D), jnp.float32),
            *[pltpu.SemaphoreType.DMA for _ in range(2 * NBUF)],
        ],
    )
    def unpack(data_hbm, len_hbm, out_hbm, idx_v, len_v, *rest):
        bufs = rest[:NBUF]
        zbuf = rest[NBUF]
        gsems = rest[NBUF + 1:2 * NBUF + 1]
        wsems = rest[2 * NBUF + 1:3 * NBUF + 1]
        wid = lax.axis_index("s") * NC + lax.axis_index("c")

        len_v[...] = jnp.zeros((L,), jnp.int32)
        pltpu.sync_copy(len_hbm, len_v.at[pl.ds(0, B)])
        lanes = lax.iota(jnp.int32, L)
        lv = len_v[...]
        lens = [lv[j] for j in range(B)]

        def meta(k):
            gg = wid + k * NW
            bk = gg // GB
            t0k = (gg % GB) * G
            lb = lens[0] * 0
            for j in range(B):
                lb = jnp.where(bk == j, lens[j], lb)
            vk = jnp.clip(lb - t0k, 0, G)
            return gg, bk, t0k, vk

        def idx_fill(k, bk, t0k):
            for s in range(G // L):
                t_vec = t0k + s * L + lanes
                acc = jnp.zeros((L,), jnp.int32)
                for lj in lens:
                    acc = acc + jnp.minimum(t_vec, lj)
                idx_v[pl.ds(k * G + s * L, L)] = jnp.minimum(
                    acc + bk, N - 1)

        def gather(k, p):
            return pltpu.make_async_copy(
                data_hbm.at[idx_v.at[pl.ds(k * G, G)]], bufs[p], gsems[p]
            )

        def write(k, p, gg, src):
            return pltpu.make_async_copy(
                src, out_hbm.at[pl.ds(gg * G, G)], wsems[p]
            )

        metas = {}
        for k in range(min(NBUF, KG)):
            metas[k] = meta(k)
            gg, bk, t0k, vk = metas[k]
            idx_fill(k, bk, t0k)

            @pl.when(vk > 0)
            def _(k=k, p=k % NBUF):
                gather(k, p).start()

        def zrow(i, carry):
            for c in range(D // L):
                zbuf[i, pl.ds(c * L, L)] = jnp.zeros((L,), jnp.float32)
            return carry

        lax.fori_loop(0, G, zrow, 0)

        for k in range(min(NBUF, KG), KG):
            metas[k] = meta(k)
            gg, bk, t0k, vk = metas[k]
            idx_fill(k, bk, t0k)

        for k in range(KG):
            p = k % NBUF
            gg, bk, t0k, vk = metas[k]

            @pl.when(vk > 0)
            def _(k=k, p=p, gg=gg, vk=vk):
                gather(k, p).wait()

                @pl.when(vk < G)
                def _():
                    def zfix(i, carry):
                        for c in range(D // L):
                            bufs[p][i, pl.ds(c * L, L)] = jnp.zeros(
                                (L,), jnp.float32)
                        return carry

                    lax.fori_loop(vk, G, zfix, 0)

                write(k, p, gg, bufs[p]).start()

            @pl.when(vk <= 0)
            def _(k=k, p=p, gg=gg):
                write(k, p, gg, zbuf).start()

            if k + NBUF < KG:
                vn = metas[k + NBUF][3]
                write(k, p, gg, bufs[p]).wait()

                @pl.when(vn > 0)
                def _(k=k, p=p, vn=vn):
                    gather(k + NBUF, p).start()

        for k in range(max(0, KG - NBUF), KG):
            p = k % NBUF
            gg = metas[k][0]
            write(k, p, gg, bufs[p]).wait()

    return unpack


def kernel(data, lengths):
    N, D = data.shape
    B = lengths.shape[0]
    out = _make_unpack(N, D, B)(data, lengths.astype(jnp.int32))
    return out.reshape(B, T_OUT, D), lengths


# speedup vs baseline: 1.0962x; 1.0346x over previous
import functools

import jax
import jax.numpy as jnp
from jax import lax
from jax.experimental import pallas as pl
from jax.experimental.pallas import tpu as pltpu
from jax.experimental.pallas import tpu_sc as plsc

T_OUT = 2048


@functools.cache
def _make_unpack(N, D, B):
    info = plsc.get_sparse_core_info()
    NC, NS, L = info.num_cores, info.num_subcores, info.num_lanes
    NW = NC * NS
    G = 32
    NGT = (B * T_OUT) // G
    KG = NGT // NW
    assert KG * NW == NGT and T_OUT % G == 0
    GB = T_OUT // G
    NBUF = 3
    ZR = 16

    mesh = plsc.VectorSubcoreMesh(core_axis_name="c", subcore_axis_name="s")

    @functools.partial(
        pl.kernel,
        mesh=mesh,
        out_type=jax.ShapeDtypeStruct((B * T_OUT, D), jnp.float32),
        scratch_types=[
            pltpu.VMEM((KG * G,), jnp.int32),
            pltpu.VMEM((L,), jnp.int32),
            *[pltpu.VMEM((G, D), jnp.float32) for _ in range(NBUF)],
            pltpu.VMEM((ZR, D), jnp.float32),
            *[pltpu.SemaphoreType.DMA for _ in range(2 * NBUF)],
        ],
    )
    def unpack(data_hbm, len_hbm, out_hbm, idx_v, len_v, *rest):
        bufs = rest[:NBUF]
        zbuf = rest[NBUF]
        gsems = rest[NBUF + 1:2 * NBUF + 1]
        wsems = rest[2 * NBUF + 1:3 * NBUF + 1]
        wid = lax.axis_index("s") * NC + lax.axis_index("c")

        len_v[...] = jnp.zeros((L,), jnp.int32)
        pltpu.sync_copy(len_hbm, len_v.at[pl.ds(0, B)])
        lanes = lax.iota(jnp.int32, L)
        lv = len_v[...]
        lens = [lv[j] for j in range(B)]

        def meta(k):
            gg = wid + k * NW
            bk = gg // GB
            t0k = (gg % GB) * G
            lb = lens[0] * 0
            for j in range(B):
                lb = jnp.where(bk == j, lens[j], lb)
            vk = jnp.clip(lb - t0k, 0, G)
            return gg, bk, t0k, vk

        def idx_fill(k, bk, t0k):
            for s in range(G // L):
                t_vec = t0k + s * L + lanes
                acc = jnp.zeros((L,), jnp.int32)
                for lj in lens:
                    acc = acc + jnp.minimum(t_vec, lj)
                idx_v[pl.ds(k * G + s * L, L)] = jnp.minimum(
                    acc + bk, N - 1)

        def gather(k, p):
            return pltpu.make_async_copy(
                data_hbm.at[idx_v.at[pl.ds(k * G, G)]], bufs[p], gsems[p]
            )

        def write(k, p, gg, src):
            return pltpu.make_async_copy(
                src, out_hbm.at[pl.ds(gg * G, G)], wsems[p]
            )

        metas = {}
        for k in range(min(NBUF, KG)):
            metas[k] = meta(k)
            gg, bk, t0k, vk = metas[k]
            idx_fill(k, bk, t0k)

            @pl.when(vk > 0)
            def _(k=k, p=k % NBUF):
                gather(k, p).start()

        def zrow(i, carry):
            for c in range(D // L):
                zbuf[i, pl.ds(c * L, L)] = jnp.zeros((L,), jnp.float32)
            return carry

        lax.fori_loop(0, ZR, zrow, 0)

        for k in range(min(NBUF, KG), KG):
            metas[k] = meta(k)
            gg, bk, t0k, vk = metas[k]
            idx_fill(k, bk, t0k)

        for k in range(KG):
            p = k % NBUF
            gg, bk, t0k, vk = metas[k]

            @pl.when(vk > 0)
            def _(k=k, p=p, gg=gg, vk=vk):
                gather(k, p).wait()

                @pl.when(vk < G)
                def _():
                    def zfix(i, carry):
                        for c in range(D // L):
                            bufs[p][i, pl.ds(c * L, L)] = jnp.zeros(
                                (L,), jnp.float32)
                        return carry

                    lax.fori_loop(vk, G, zfix, 0)

                write(k, p, gg, bufs[p]).start()

            @pl.when(vk <= 0)
            def _(k=k, p=p, gg=gg):
                for q in range(G // ZR):
                    pltpu.make_async_copy(
                        zbuf,
                        out_hbm.at[pl.ds(gg * G + q * ZR, ZR)],
                        wsems[p],
                    ).start()

            if k + NBUF < KG:
                vn = metas[k + NBUF][3]
                write(k, p, gg, bufs[p]).wait()

                @pl.when(vn > 0)
                def _(k=k, p=p, vn=vn):
                    gather(k + NBUF, p).start()

        for k in range(max(0, KG - NBUF), KG):
            p = k % NBUF
            gg = metas[k][0]
            write(k, p, gg, bufs[p]).wait()

    return unpack


def kernel(data, lengths):
    N, D = data.shape
    B = lengths.shape[0]
    out = _make_unpack(N, D, B)(data, lengths.astype(jnp.int32))
    return out.reshape(B, T_OUT, D), lengths


# speedup vs baseline: 1.1691x; 1.0665x over previous
import functools

import jax
import jax.numpy as jnp
from jax import lax
from jax.experimental import pallas as pl
from jax.experimental.pallas import tpu as pltpu
from jax.experimental.pallas import tpu_sc as plsc

T_OUT = 2048


@functools.cache
def _make_unpack(N, D, B):
    info = plsc.get_sparse_core_info()
    NC, NS, L = info.num_cores, info.num_subcores, info.num_lanes
    NW = NC * NS
    G = 32
    NGT = (B * T_OUT) // G
    KG = NGT // NW
    assert KG * NW == NGT and T_OUT % G == 0
    GB = T_OUT // G
    NBUF = 3
    ZR = 16

    mesh = plsc.VectorSubcoreMesh(core_axis_name="c", subcore_axis_name="s")

    @functools.partial(
        pl.kernel,
        mesh=mesh,
        out_type=jax.ShapeDtypeStruct((B * T_OUT, D), jnp.float32),
        scratch_types=[
            pltpu.VMEM((KG * G,), jnp.int32),
            pltpu.VMEM((L,), jnp.int32),
            *[pltpu.VMEM((G, D), jnp.float32) for _ in range(NBUF)],
            pltpu.VMEM((ZR, D), jnp.float32),
            *[pltpu.SemaphoreType.DMA for _ in range(2 * NBUF)],
        ],
    )
    def unpack(data_hbm, len_hbm, out_hbm, idx_v, len_v, *rest):
        bufs = rest[:NBUF]
        zbuf = rest[NBUF]
        gsems = rest[NBUF + 1:2 * NBUF + 1]
        wsems = rest[2 * NBUF + 1:3 * NBUF + 1]
        wid = lax.axis_index("s") * NC + lax.axis_index("c")

        len_v[...] = jnp.zeros((L,), jnp.int32)
        pltpu.sync_copy(len_hbm, len_v.at[pl.ds(0, B)])
        lanes = lax.iota(jnp.int32, L)
        lv = len_v[...]
        lens = [lv[j] for j in range(B)]

        def meta(k):
            gg = wid + k * NW
            bk = gg // GB
            t0k = (gg % GB) * G
            lb = lens[0] * 0
            for j in range(B):
                lb = jnp.where(bk == j, lens[j], lb)
            vk = jnp.clip(lb - t0k, 0, G)
            return gg, bk, t0k, vk

        def idx_fill(k, bk, t0k):
            for s in range(G // L):
                t_vec = t0k + s * L + lanes
                acc = jnp.zeros((L,), jnp.int32)
                for lj in lens:
                    acc = acc + jnp.minimum(t_vec, lj)
                idx_v[pl.ds(k * G + s * L, L)] = jnp.minimum(
                    acc + bk, N - 1)

        def gather(k, p):
            return pltpu.make_async_copy(
                data_hbm.at[idx_v.at[pl.ds(k * G, G)]], bufs[p], gsems[p]
            )

        def write(gg, p, src):
            return pltpu.make_async_copy(
                src, out_hbm.at[pl.ds(gg * G, G)], wsems[p]
            )

        for k in range(min(NBUF, KG)):
            gg, bk, t0k, vk = meta(k)
            idx_fill(k, bk, t0k)

            @pl.when(vk > 0)
            def _(k=k, p=k % NBUF):
                gather(k, p).start()

        def zrow(i, carry):
            for c in range(D // L):
                zbuf[i, pl.ds(c * L, L)] = jnp.zeros((L,), jnp.float32)
            return carry

        lax.fori_loop(0, ZR, zrow, 0)

        def idx_body(k, carry):
            _, bk, t0k, _ = meta(k)
            idx_fill(k, bk, t0k)
            return carry

        lax.fori_loop(min(NBUF, KG), KG, idx_body, 0)

        def slot(k, p):
            gg, bk, t0k, vk = meta(k)

            @pl.when(vk > 0)
            def _():
                gather(k, p).wait()

                @pl.when(vk < G)
                def _():
                    def zfix(i, carry):
                        for c in range(D // L):
                            bufs[p][i, pl.ds(c * L, L)] = jnp.zeros(
                                (L,), jnp.float32)
                        return carry

                    lax.fori_loop(vk, G, zfix, 0)

                write(gg, p, bufs[p]).start()

            @pl.when(vk <= 0)
            def _():
                for q in range(G // ZR):
                    pltpu.make_async_copy(
                        zbuf,
                        out_hbm.at[pl.ds(gg * G + q * ZR, ZR)],
                        wsems[p],
                    ).start()

            @pl.when(k + NBUF < KG)
            def _():
                vn = meta(k + NBUF)[3]
                write(gg, p, bufs[p]).wait()

                @pl.when(vn > 0)
                def _():
                    gather(k + NBUF, p).start()

        def main_body(j, carry):
            for i in range(NBUF):
                k = j * NBUF + i

                @pl.when(k < KG)
                def _(k=k, i=i):
                    slot(k, i)

            return carry

        lax.fori_loop(0, (KG + NBUF - 1) // NBUF, main_body, 0)

        for k in range(max(0, KG - NBUF), KG):
            gg = meta(k)[0]
            write(gg, k % NBUF, bufs[k % NBUF]).wait()

    return unpack


def kernel(data, lengths):
    N, D = data.shape
    B = lengths.shape[0]
    out = _make_unpack(N, D, B)(data, lengths.astype(jnp.int32))
    return out.reshape(B, T_OUT, D), lengths


# speedup vs baseline: 1.1845x; 1.0132x over previous
import functools

import jax
import jax.numpy as jnp
from jax import lax
from jax.experimental import pallas as pl
from jax.experimental.pallas import tpu as pltpu
from jax.experimental.pallas import tpu_sc as plsc

T_OUT = 2048


@functools.cache
def _make_unpack(N, D, B):
    info = plsc.get_sparse_core_info()
    NC, NS, L = info.num_cores, info.num_subcores, info.num_lanes
    NW = NC * NS
    G = 32
    NGT = (B * T_OUT) // G
    KG = NGT // NW
    assert KG * NW == NGT and T_OUT % G == 0
    GB = T_OUT // G
    NBUF = 3
    ZR = 16

    mesh = plsc.VectorSubcoreMesh(core_axis_name="c", subcore_axis_name="s")

    @functools.partial(
        pl.kernel,
        mesh=mesh,
        out_type=jax.ShapeDtypeStruct((B * T_OUT, D), jnp.float32),
        scratch_types=[
            pltpu.VMEM((KG * G,), jnp.int32),
            pltpu.VMEM((L,), jnp.int32),
            *[pltpu.VMEM((G, D), jnp.float32) for _ in range(NBUF)],
            pltpu.VMEM((ZR, D), jnp.float32),
            *[pltpu.SemaphoreType.DMA for _ in range(2 * NBUF)],
        ],
    )
    def unpack(data_hbm, len_hbm, out_hbm, idx_v, len_v, *rest):
        bufs = rest[:NBUF]
        zbuf = rest[NBUF]
        gsems = rest[NBUF + 1:2 * NBUF + 1]
        wsems = rest[2 * NBUF + 1:3 * NBUF + 1]
        wid = lax.axis_index("s") * NC + lax.axis_index("c")

        len_v[...] = jnp.zeros((L,), jnp.int32)
        pltpu.sync_copy(len_hbm, len_v.at[pl.ds(0, B)])
        lanes = lax.iota(jnp.int32, L)
        lv = len_v[...]
        lens = [lv[j] for j in range(B)]

        def meta(k):
            gg = wid + k * NW
            bk = gg // GB
            t0k = (gg % GB) * G
            lb = lens[0] * 0
            for j in range(B):
                lb = jnp.where(bk == j, lens[j], lb)
            vk = jnp.clip(lb - t0k, 0, G)
            return gg, bk, t0k, vk

        def idx_fill(k, bk, t0k):
            for s in range(G // L):
                t_vec = t0k + s * L + lanes
                acc = jnp.zeros((L,), jnp.int32)
                for lj in lens:
                    acc = acc + jnp.minimum(t_vec, lj)
                idx_v[pl.ds(k * G + s * L, L)] = jnp.minimum(
                    acc + bk, N - 1)

        def gather(k, p):
            return pltpu.make_async_copy(
                data_hbm.at[idx_v.at[pl.ds(k * G, G)]], bufs[p], gsems[p]
            )

        def write(gg, p, src):
            return pltpu.make_async_copy(
                src, out_hbm.at[pl.ds(gg * G, G)], wsems[p]
            )

        for k in range(min(NBUF, KG)):
            gg, bk, t0k, vk = meta(k)
            idx_fill(k, bk, t0k)

            @pl.when(vk > 0)
            def _(k=k, p=k % NBUF):
                gather(k, p).start()

        def zrow(i, carry):
            def zcol(c, carry2):
                zbuf[i, pl.ds(c * L, L)] = jnp.zeros((L,), jnp.float32)
                return carry2

            return lax.fori_loop(0, D // L, zcol, carry)

        lax.fori_loop(0, ZR, zrow, 0)

        def idx_body(k, carry):
            _, bk, t0k, _ = meta(k)
            idx_fill(k, bk, t0k)
            return carry

        lax.fori_loop(min(NBUF, KG), KG, idx_body, 0)

        def slot(k, p):
            gg, bk, t0k, vk = meta(k)

            @pl.when(vk > 0)
            def _():
                gather(k, p).wait()

                @pl.when(vk < G)
                def _():
                    def zfix(i, carry):
                        def zcol(c, carry2):
                            bufs[p][i, pl.ds(c * L, L)] = jnp.zeros(
                                (L,), jnp.float32)
                            return carry2

                        return lax.fori_loop(0, D // L, zcol, carry)

                    lax.fori_loop(vk, G, zfix, 0)

                write(gg, p, bufs[p]).start()

            @pl.when(vk <= 0)
            def _():
                for q in range(G // ZR):
                    pltpu.make_async_copy(
                        zbuf,
                        out_hbm.at[pl.ds(gg * G + q * ZR, ZR)],
                        wsems[p],
                    ).start()

            @pl.when(k + NBUF < KG)
            def _():
                vn = meta(k + NBUF)[3]
                write(gg, p, bufs[p]).wait()

                @pl.when(vn > 0)
                def _():
                    gather(k + NBUF, p).start()

        def main_body(j, carry):
            for i in range(NBUF):
                k = j * NBUF + i

                @pl.when(k < KG)
                def _(k=k, i=i):
                    slot(k, i)

            return carry

        lax.fori_loop(0, (KG + NBUF - 1) // NBUF, main_body, 0)

        for k in range(max(0, KG - NBUF), KG):
            write(wid + k * NW, k % NBUF, bufs[k % NBUF]).wait()

    return unpack


def kernel(data, lengths):
    N, D = data.shape
    B = lengths.shape[0]
    out = _make_unpack(N, D, B)(data, lengths.astype(jnp.int32))
    return out.reshape(B, T_OUT, D), lengths
